# baseline (speedup 1.0000x reference)
import jax
import jax.numpy as jnp
from jax import lax
from jax.experimental import pallas as pl
from jax.experimental.pallas import tpu as pltpu
from jax.experimental.pallas import tpu_sc as plsc

_N_CORES = 2
_N_SUBCORES = 16
_N_WORKERS = _N_CORES * _N_SUBCORES
_CHUNK = 4096


def _sc_body(idx_hbm, table_hbm, out_hbm, idx_v, rows_v, sem):
    wid = lax.axis_index("s") * _N_CORES + lax.axis_index("c")
    n = idx_hbm.shape[0]
    b_per_w = n // _N_WORKERS
    base = wid * b_per_w
    n_chunks = b_per_w // _CHUNK
    for i in range(n_chunks):
        off = base + i * _CHUNK
        pltpu.sync_copy(idx_hbm.at[pl.ds(off, _CHUNK)], idx_v)
        pltpu.async_copy(table_hbm.at[idx_v], rows_v, sem).wait()
        pltpu.sync_copy(rows_v, out_hbm.at[pl.ds(off, _CHUNK)])


def kernel(inputs, table):
    b, s = inputs.shape
    n = b * s
    d = table.shape[1]
    idx = inputs.reshape(n)
    mesh = plsc.VectorSubcoreMesh(core_axis_name="c", subcore_axis_name="s")
    k = pl.kernel(
        _sc_body,
        out_type=jax.ShapeDtypeStruct((n, d), jnp.float32),
        mesh=mesh,
        scratch_types=[
            pltpu.VMEM((_CHUNK,), jnp.int32),
            pltpu.VMEM((_CHUNK, d), jnp.float32),
            pltpu.SemaphoreType.DMA,
        ],
        compiler_params=pltpu.CompilerParams(use_tc_tiling_on_sc=False),
    )
    out = k(idx, table)
    return out.reshape(b, s, d)


# speedup vs baseline: 7.7870x; 7.7870x over previous
import jax
import jax.numpy as jnp
from jax import lax
from jax.experimental import pallas as pl
from jax.experimental.pallas import tpu as pltpu
from jax.experimental.pallas import tpu_sc as plsc

_N_CORES = 2
_N_SUBCORES = 16
_N_WORKERS = _N_CORES * _N_SUBCORES
_CHUNK = 4096
_LANES = 16
_D = 8


def _sc_body(idx_hbm, table_hbm, out_hbm, table_v, idx_v0, idx_v1, out_v0,
             out_v1, sem_i0, sem_i1, sem_o0, sem_o1):
    wid = lax.axis_index("s") * _N_CORES + lax.axis_index("c")
    n = idx_hbm.shape[0]
    b_per_w = n // _N_WORKERS
    base = wid * b_per_w
    n_chunks = b_per_w // _CHUNK

    pltpu.sync_copy(table_hbm, table_v)

    idx_bufs = [idx_v0, idx_v1]
    out_bufs = [out_v0, out_v1]
    idx_sems = [sem_i0, sem_i1]
    out_sems = [sem_o0, sem_o1]

    lane = lax.iota(jnp.int32, _LANES)
    row_base = lane * _D

    def compute_chunk(slot):
        idx_buf = idx_bufs[slot]
        out_buf = out_bufs[slot]

        def body(j, carry):
            idxv = idx_buf[pl.ds(j * _LANES, _LANES)]
            addr = idxv * _D
            out_base = row_base + j * (_LANES * _D)
            for k in range(_D):
                vals = plsc.load_gather(table_v, [addr + k])
                plsc.store_scatter(out_buf, [out_base + k], vals)
            return carry

        lax.fori_loop(0, _CHUNK // _LANES, body, 0, unroll=2)

    idx_copies = [None, None]
    out_copies = [None, None]
    for i in range(min(2, n_chunks)):
        idx_copies[i] = pltpu.async_copy(
            idx_hbm.at[pl.ds(base + i * _CHUNK, _CHUNK)], idx_bufs[i],
            idx_sems[i])

    for i in range(n_chunks):
        slot = i % 2
        idx_copies[slot].wait()
        if out_copies[slot] is not None:
            out_copies[slot].wait()
        compute_chunk(slot)
        out_copies[slot] = pltpu.async_copy(
            out_bufs[slot],
            out_hbm.at[pl.ds((base + i * _CHUNK) * _D, _CHUNK * _D)],
            out_sems[slot])
        if i + 2 < n_chunks:
            idx_copies[slot] = pltpu.async_copy(
                idx_hbm.at[pl.ds(base + (i + 2) * _CHUNK, _CHUNK)],
                idx_bufs[slot], idx_sems[slot])

    for c in out_copies:
        if c is not None:
            c.wait()


def kernel(inputs, table):
    b, s = inputs.shape
    n = b * s
    d = table.shape[1]
    idx = inputs.reshape(n)
    table_flat = table.reshape(d * d)
    mesh = plsc.VectorSubcoreMesh(core_axis_name="c", subcore_axis_name="s")
    k = pl.kernel(
        _sc_body,
        out_type=jax.ShapeDtypeStruct((n * d,), jnp.float32),
        mesh=mesh,
        scratch_types=[
            pltpu.VMEM((d * d,), jnp.float32),
            pltpu.VMEM((_CHUNK,), jnp.int32),
            pltpu.VMEM((_CHUNK,), jnp.int32),
            pltpu.VMEM((_CHUNK * d,), jnp.float32),
            pltpu.VMEM((_CHUNK * d,), jnp.float32),
            pltpu.SemaphoreType.DMA,
            pltpu.SemaphoreType.DMA,
            pltpu.SemaphoreType.DMA,
            pltpu.SemaphoreType.DMA,
        ],
        compiler_params=pltpu.CompilerParams(needs_layout_passes=False),
    )
    out = k(idx, table_flat)
    return out.reshape(b, s, d)
